# Initial kernel scaffold; baseline (speedup 1.0000x reference)
import jax
import jax.numpy as jnp
from jax.experimental import pallas as pl


def kernel(x, edge_index, batch, W1, b1, Wc1, bc1, Wc2, bc2, Wc3, bc3):
    raise NotImplementedError("write your pallas kernel here")


# speedup vs baseline: 4.7859x; 4.7859x over previous
import functools
import math

import jax
import jax.numpy as jnp
from jax.experimental import pallas as pl
from jax.experimental.pallas import tpu as pltpu

N = 10000
D = 256
H = 256
G = 64
OUT = 2
NEG_INF = float("-inf")


def _fused_kernel(x_ref, batch_col_ref, batch_row_ref,
                  w1t_ref, b1_ref, wc1t_ref, bc1_ref,
                  wc2t_ref, bc2_ref, wc3t_ref, bc3_ref,
                  out_ref):
    f32 = jnp.float32

    h = jnp.dot(x_ref[...], w1t_ref[...], preferred_element_type=f32)
    h = h + b1_ref[...]

    batch_col = batch_col_ref[...]
    batch_row = batch_row_ref[...]

    seg_iota = jax.lax.broadcasted_iota(jnp.int32, (G, 1), 0)
    eq = (batch_row == seg_iota).astype(f32)
    le = (batch_row <= seg_iota).astype(f32)

    counts = jnp.sum(eq, axis=1, keepdims=True)
    ends = jnp.sum(le, axis=1, keepdims=True).astype(jnp.int32) - 1

    x_sum = jnp.dot(eq, h, preferred_element_type=f32)

    m = h
    ids = batch_col
    steps = int(math.ceil(math.log2(N)))
    for k in range(steps):
        s = 1 << k
        m_sh = jnp.concatenate(
            [jnp.full((s, H), NEG_INF, dtype=f32), m[: N - s, :]], axis=0)
        ids_sh = jnp.concatenate(
            [jnp.full((s, 1), -1, dtype=jnp.int32), ids[: N - s, :]], axis=0)
        same = ids_sh == ids
        m = jnp.maximum(m, jnp.where(same, m_sh, NEG_INF))

    col_iota = jax.lax.broadcasted_iota(jnp.int32, (1, N), 1)
    sel = ((col_iota == ends) & (counts > 0.0)).astype(f32)
    gathered = jnp.dot(sel, m, preferred_element_type=f32)
    x_max = jnp.where(counts > 0.0, gathered, NEG_INF)

    x_mean = x_sum / jnp.maximum(counts, 1.0)

    x_global = jnp.concatenate([x_mean, x_max, x_sum], axis=1)

    z = jnp.dot(x_global, wc1t_ref[...], preferred_element_type=f32)
    z = jnp.maximum(z + bc1_ref[...], 0.0)
    z = jnp.dot(z, wc2t_ref[...], preferred_element_type=f32)
    z = jnp.maximum(z + bc2_ref[...], 0.0)
    z = jnp.dot(z, wc3t_ref[...], preferred_element_type=f32)
    out_ref[...] = z + bc3_ref[...]


@jax.jit
def _run(x, batch, W1, b1, Wc1, bc1, Wc2, bc2, Wc3, bc3):
    batch_col = batch.reshape(N, 1)
    batch_row = batch.reshape(1, N)
    wc3t_pad = jnp.zeros((H // 2, 128), jnp.float32).at[:, :OUT].set(Wc3.T)
    bc3_pad = jnp.zeros((1, 128), jnp.float32).at[:, :OUT].set(bc3)
    out = pl.pallas_call(
        _fused_kernel,
        out_shape=jax.ShapeDtypeStruct((G, 128), jnp.float32),
    )(x, batch_col, batch_row,
      W1.T, b1.reshape(1, H), Wc1.T, bc1.reshape(1, H),
      Wc2.T, bc2.reshape(1, H // 2), wc3t_pad, bc3_pad)
    return out[:, :OUT]


def kernel(x, edge_index, batch, W1, b1, Wc1, bc1, Wc2, bc2, Wc3, bc3):
    del edge_index
    return _run(x, batch, W1, b1, Wc1, bc1, Wc2, bc2, Wc3, bc3)
